# baseline (speedup 1.0000x reference)
import functools
import math

import jax
import jax.numpy as jnp
from jax import lax
from jax.experimental import pallas as pl
from jax.experimental.pallas import tpu as pltpu
from jax.experimental.pallas import tpu_sc as plsc

D_MODEL = 64
SCALE = math.sqrt(D_MODEL)

NUM_CORES = 2
NUM_SUBCORES = 16
NUM_WORKERS = NUM_CORES * NUM_SUBCORES
CHUNK = 128
LANES = 16


def _embed_lookup(idx_flat, table):
    n_idx = idx_flat.shape[0]
    b_per_w = n_idx // NUM_WORKERS
    n_chunks = b_per_w // CHUNK
    mesh = plsc.VectorSubcoreMesh(core_axis_name="c", subcore_axis_name="s")

    @functools.partial(
        pl.kernel,
        mesh=mesh,
        out_type=jax.ShapeDtypeStruct((n_idx, D_MODEL), jnp.float32),
        scratch_types=[
            pltpu.VMEM((CHUNK,), jnp.int32),
            pltpu.VMEM((CHUNK, D_MODEL), jnp.float32),
            pltpu.SemaphoreType.DMA,
        ],
        compiler_params=pltpu.CompilerParams(use_tc_tiling_on_sc=False),
    )
    def body(idx_hbm, table_hbm, out_hbm, idx_v, rows_v, sem):
        wid = lax.axis_index("s") * NUM_CORES + lax.axis_index("c")
        base = wid * b_per_w

        def chunk_body(i, carry):
            off = base + i * CHUNK
            pltpu.sync_copy(idx_hbm.at[pl.ds(off, CHUNK)], idx_v)
            pltpu.async_copy(table_hbm.at[idx_v], rows_v, sem).wait()

            def row_body(r, c):
                for j in range(D_MODEL // LANES):
                    sl = pl.ds(j * LANES, LANES)
                    rows_v[r, sl] = rows_v[r, sl] * SCALE
                return c

            lax.fori_loop(0, CHUNK, row_body, 0)
            pltpu.sync_copy(rows_v, out_hbm.at[pl.ds(off, CHUNK)])
            return carry

        lax.fori_loop(0, n_chunks, chunk_body, 0)

    return body(idx_flat, table)


def kernel(x, table):
    s0, s1 = x.shape
    idx_flat = x.reshape(s0 * s1)
    out = _embed_lookup(idx_flat, table)
    return out.reshape(s0, s1, D_MODEL)


# speedup vs baseline: 1.2664x; 1.2664x over previous
import functools
import math

import jax
import jax.numpy as jnp
from jax import lax
from jax.experimental import pallas as pl
from jax.experimental.pallas import tpu as pltpu
from jax.experimental.pallas import tpu_sc as plsc

D_MODEL = 64
SCALE = math.sqrt(D_MODEL)

NUM_CORES = 2
NUM_SUBCORES = 16
NUM_WORKERS = NUM_CORES * NUM_SUBCORES
CHUNK = 128
NBUF = 8
LANES = 16
ROWS_PER_IT = 4


def _embed_lookup(idx_flat, table):
    n_idx = idx_flat.shape[0]
    b_per_w = n_idx // NUM_WORKERS
    n_groups = b_per_w // (CHUNK * NBUF)
    mesh = plsc.VectorSubcoreMesh(core_axis_name="c", subcore_axis_name="s")

    @functools.partial(
        pl.kernel,
        mesh=mesh,
        out_type=jax.ShapeDtypeStruct((n_idx, D_MODEL), jnp.float32),
        scratch_types=[
            pltpu.VMEM((NBUF, CHUNK), jnp.int32),
            pltpu.VMEM((NBUF, CHUNK, D_MODEL), jnp.float32),
            [pltpu.SemaphoreType.DMA] * NBUF,
            [pltpu.SemaphoreType.DMA] * NBUF,
            pltpu.SemaphoreType.DMA,
        ],
        compiler_params=pltpu.CompilerParams(use_tc_tiling_on_sc=False),
    )
    def body(idx_hbm, table_hbm, out_hbm, idx_v, rows_v, sem_i, sem_g, sem_s):
        wid = lax.axis_index("s") * NUM_CORES + lax.axis_index("c")
        base = wid * b_per_w

        def group_body(g, carry):
            g0 = base + g * (CHUNK * NBUF)
            idx_handles = [
                pltpu.async_copy(
                    idx_hbm.at[pl.ds(g0 + b * CHUNK, CHUNK)], idx_v.at[b], sem_i[b]
                )
                for b in range(NBUF)
            ]
            gather_handles = []
            for b in range(NBUF):
                idx_handles[b].wait()
                gather_handles.append(
                    pltpu.async_copy(table_hbm.at[idx_v.at[b]], rows_v.at[b], sem_g[b])
                )
            out_handles = []
            for b in range(NBUF):
                gather_handles[b].wait()

                def scale_body(r, c, b=b):
                    for k in range(ROWS_PER_IT):
                        for j in range(D_MODEL // LANES):
                            sl = pl.ds(j * LANES, LANES)
                            row = r * ROWS_PER_IT + k
                            rows_v[b, row, sl] = rows_v[b, row, sl] * SCALE
                    return c

                lax.fori_loop(0, CHUNK // ROWS_PER_IT, scale_body, 0)
                out_handles.append(
                    pltpu.async_copy(
                        rows_v.at[b], out_hbm.at[pl.ds(g0 + b * CHUNK, CHUNK)], sem_s
                    )
                )
            for h in out_handles:
                h.wait()
            return carry

        lax.fori_loop(0, n_groups, group_body, 0)

    return body(idx_flat, table)


def kernel(x, table):
    s0, s1 = x.shape
    idx_flat = x.reshape(s0 * s1)
    out = _embed_lookup(idx_flat, table)
    return out.reshape(s0, s1, D_MODEL)
